# baseline (speedup 1.0000x reference)
import jax
import jax.numpy as jnp
from jax.experimental import pallas as pl
from jax.experimental.pallas import tpu as pltpu

E = 8
TOPK = 2
H = 1024
FF = 4096
T = 16
HB = 128
FB = 1024
NH = H // HB
NF = FF // FB


def _moe_kernel(x_ref, rw_ref, gup_ref, wd_ref,
                out_ref, logits_ref, scores_ref, acc_ref):
    e = pl.program_id(0)
    i = pl.program_id(1)

    @pl.when(jnp.logical_and(e == 0, i == 0))
    def _prologue():
        x = x_ref[...]
        logits = jnp.dot(x, rw_ref[...].T, preferred_element_type=jnp.float32)
        logits_ref[...] = logits
        idx = jax.lax.broadcasted_iota(jnp.int32, (T, E), 1)
        m1 = jnp.max(logits, axis=1, keepdims=True)
        i1 = jnp.min(jnp.where(logits == m1, idx, E), axis=1, keepdims=True)
        mask1 = idx == i1
        l2 = jnp.where(mask1, -jnp.inf, logits)
        m2 = jnp.max(l2, axis=1, keepdims=True)
        i2 = jnp.min(jnp.where(l2 == m2, idx, E), axis=1, keepdims=True)
        mask2 = idx == i2
        scores_ref[...] = jnp.where(mask1 | mask2,
                                    jax.nn.sigmoid(logits), 0.0)
        out_ref[...] = jnp.zeros_like(out_ref)

    @pl.when(i < NH)
    def _phase_a():
        col = jax.lax.broadcasted_iota(jnp.int32, (T, E), 1)
        s = jnp.sum(jnp.where(col == e, scores_ref[...], 0.0), axis=1,
                    keepdims=True)
        xs = x_ref[:, pl.ds(i * HB, HB)] * s

        @pl.when(i == 0)
        def _zero_acc():
            acc_ref[...] = jnp.zeros_like(acc_ref)

        acc_ref[...] += jnp.dot(xs, gup_ref[0],
                                preferred_element_type=jnp.float32)

    @pl.when(i >= NH)
    def _phase_b():
        fj = i - NH
        gate = acc_ref[:, pl.ds(fj * FB, FB)]
        up = acc_ref[:, pl.ds(FF + fj * FB, FB)]
        act = up * (gate * jax.nn.sigmoid(gate))
        out_ref[...] += jnp.dot(act, wd_ref[0],
                                preferred_element_type=jnp.float32)


def _moe(hidden_states, router_weight, gate_up_proj, down_proj,
         interpret=False):
    out, logits = pl.pallas_call(
        _moe_kernel,
        grid=(E, NH + NF),
        in_specs=[
            pl.BlockSpec((T, H), lambda e, i: (0, 0)),
            pl.BlockSpec((E, H), lambda e, i: (0, 0)),
            pl.BlockSpec((1, HB, 2 * FF),
                         lambda e, i: (e, jnp.minimum(i, NH - 1), 0)),
            pl.BlockSpec((1, FB, H),
                         lambda e, i: (e, jnp.maximum(i - NH, 0), 0)),
        ],
        out_specs=[
            pl.BlockSpec((T, H), lambda e, i: (0, 0)),
            pl.BlockSpec((T, E), lambda e, i: (0, 0)),
        ],
        out_shape=[
            jax.ShapeDtypeStruct((T, H), jnp.float32),
            jax.ShapeDtypeStruct((T, E), jnp.float32),
        ],
        scratch_shapes=[pltpu.VMEM((T, E), jnp.float32),
                        pltpu.VMEM((T, 2 * FF), jnp.float32)],
        interpret=interpret,
    )(hidden_states, router_weight, gate_up_proj, down_proj)
    return out, logits


def kernel(hidden_states, router_weight, gate_up_proj, down_proj):
    return _moe(hidden_states.reshape(-1, H), router_weight,
                gate_up_proj, down_proj)


# speedup vs baseline: 1.0294x; 1.0294x over previous
import jax
import jax.numpy as jnp
from jax.experimental import pallas as pl
from jax.experimental.pallas import tpu as pltpu
from jax.experimental.pallas import tpu_sc as plsc

E = 8
TOPK = 2
H = 1024
FF = 4096
T = 16
FB = 1024


def _router_sc(x_hbm, rw_hbm, logits_hbm, scores_hbm, xv, rwv, lv, sv):
    c = jax.lax.axis_index("c")
    s = jax.lax.axis_index("s")

    idx = jax.lax.iota(jnp.int32, 16)

    def _perm(v, p):
        dnums = jax.lax.GatherDimensionNumbers(
            offset_dims=(), collapsed_slice_dims=(0,), start_index_map=(0,))
        return jax.lax.gather(
            v, p[:, None], dnums, (1,),
            mode=jax.lax.GatherScatterMode.PROMISE_IN_BOUNDS)

    def _allred(v, op):
        for k in (8, 4, 2, 1):
            v = op(v, _perm(v, idx ^ k))
        return v

    def _rnd_bf16(v):
        u = jax.lax.bitcast_convert_type(v, jnp.uint32)
        u = ((u + jnp.uint32(0x7FFF) + ((u >> jnp.uint32(16))
                                        & jnp.uint32(1)))
             & jnp.uint32(0xFFFF0000))
        return jax.lax.bitcast_convert_type(u, jnp.float32)

    @pl.when(s < T // 2)
    def _work():
        t = c * (T // 2) + s
        pltpu.sync_copy(x_hbm.at[t], xv)
        pltpu.sync_copy(rw_hbm, rwv)

        def body(i, accs):
            xc = _rnd_bf16(xv[pl.ds(i * 16, 16)])
            return tuple(
                accs[e] + xc * _rnd_bf16(rwv[e, pl.ds(i * 16, 16)])
                for e in range(E))
        accs = jax.lax.fori_loop(
            0, H // 16, body,
            tuple(jnp.zeros((16,), jnp.float32) for _ in range(E)))
        logits = jnp.full((16,), -jnp.inf, jnp.float32)
        for e in range(E):
            logits = jnp.where(idx == e, _allred(accs[e], jnp.add), logits)
        m1 = _allred(logits, jnp.maximum)
        i1 = _allred(jnp.where(logits == m1, idx, 16), jnp.minimum)
        mask1 = idx == i1
        l2 = jnp.where(mask1, -jnp.inf, logits)
        m2 = _allred(l2, jnp.maximum)
        i2 = _allred(jnp.where(l2 == m2, idx, 16), jnp.minimum)
        mask = mask1 | (idx == i2)
        sig = 1.0 / (1.0 + jnp.exp(-logits))
        scores = jnp.where(mask, sig, 0.0)
        lv[...] = logits
        sv[...] = scores
        pltpu.sync_copy(lv.at[pl.ds(0, E)], logits_hbm.at[pl.ds(t * E, E)])
        pltpu.sync_copy(sv.at[pl.ds(0, E)], scores_hbm.at[pl.ds(t * E, E)])


def _router(x, rw):
    return pl.kernel(
        _router_sc,
        out_type=[jax.ShapeDtypeStruct((T * E,), jnp.float32),
                  jax.ShapeDtypeStruct((T * E,), jnp.float32)],
        mesh=plsc.VectorSubcoreMesh(core_axis_name="c", subcore_axis_name="s"),
        scratch_types=[pltpu.VMEM((H,), jnp.float32),
                       pltpu.VMEM((E, H), jnp.float32),
                       pltpu.VMEM((16,), jnp.float32),
                       pltpu.VMEM((16,), jnp.float32)],
    )(x, rw)


def _ffn_kernel(x_ref, sc_ref, wg_ref, wu_ref, wd_ref, out_ref):
    e = pl.program_id(0)
    f = pl.program_id(1)

    @pl.when(jnp.logical_and(e == 0, f == 0))
    def _init():
        out_ref[...] = jnp.zeros_like(out_ref)

    col = jax.lax.broadcasted_iota(jnp.int32, (T, E), 1)
    s = jnp.sum(jnp.where(col == e, sc_ref[...], 0.0), axis=1, keepdims=True)
    xs = x_ref[...] * s
    gate = jnp.dot(xs, wg_ref[0], preferred_element_type=jnp.float32)
    up = jnp.dot(xs, wu_ref[0], preferred_element_type=jnp.float32)
    act = up * (gate * jax.nn.sigmoid(gate))
    out_ref[...] += jnp.dot(act, wd_ref[0], preferred_element_type=jnp.float32)


def _ffn(x, scores, gate_up_proj, down_proj):
    nf = FF // FB
    return pl.pallas_call(
        _ffn_kernel,
        grid=(E, nf),
        in_specs=[
            pl.BlockSpec((T, H), lambda e, f: (0, 0)),
            pl.BlockSpec((T, E), lambda e, f: (0, 0)),
            pl.BlockSpec((1, H, FB), lambda e, f: (e, 0, f)),
            pl.BlockSpec((1, H, FB), lambda e, f: (e, 0, f + FF // FB)),
            pl.BlockSpec((1, FB, H), lambda e, f: (e, f, 0)),
        ],
        out_specs=pl.BlockSpec((T, H), lambda e, f: (0, 0)),
        out_shape=jax.ShapeDtypeStruct((T, H), jnp.float32),
    )(x, scores, gate_up_proj, gate_up_proj, down_proj)


def kernel(hidden_states, router_weight, gate_up_proj, down_proj):
    x = hidden_states.reshape(-1, H)
    logits, scores = _router(x, router_weight)
    logits = logits.reshape(T, E)
    scores = scores.reshape(T, E)
    out = _ffn(x, scores, gate_up_proj, down_proj)
    return out, logits


# speedup vs baseline: 1.2129x; 1.1782x over previous
import functools

import jax
import jax.numpy as jnp
from jax.experimental import pallas as pl
from jax.experimental.pallas import tpu as pltpu

E = 8
TOPK = 2
H = 1024
FF = 4096
T = 16
FB = 1024


def _moe_kernel(x_ref, rw_ref, wg_ref, wu_ref, wd_ref,
                out_ref, logits_ref, scores_ref):
    e = pl.program_id(0)
    f = pl.program_id(1)

    @pl.when(jnp.logical_and(e == 0, f == 0))
    def _prologue():
        x = x_ref[...]
        logits = jnp.dot(x, rw_ref[...].T, preferred_element_type=jnp.float32)
        logits_ref[...] = logits
        idx = jax.lax.broadcasted_iota(jnp.int32, (T, E), 1)
        m1 = jnp.max(logits, axis=1, keepdims=True)
        i1 = jnp.min(jnp.where(logits == m1, idx, E), axis=1, keepdims=True)
        mask1 = idx == i1
        l2 = jnp.where(mask1, -jnp.inf, logits)
        m2 = jnp.max(l2, axis=1, keepdims=True)
        i2 = jnp.min(jnp.where(l2 == m2, idx, E), axis=1, keepdims=True)
        mask2 = idx == i2
        scores_ref[...] = jnp.where(mask1 | mask2,
                                    jax.nn.sigmoid(logits), 0.0)
        out_ref[...] = jnp.zeros_like(out_ref)

    col = jax.lax.broadcasted_iota(jnp.int32, (T, E), 1)
    s = jnp.sum(jnp.where(col == e, scores_ref[...], 0.0), axis=1,
                keepdims=True)
    xs = x_ref[...] * s
    gate = jnp.dot(xs, wg_ref[0], preferred_element_type=jnp.float32)
    up = jnp.dot(xs, wu_ref[0], preferred_element_type=jnp.float32)
    act = up * (gate * jax.nn.sigmoid(gate))
    out_ref[...] += jnp.dot(act, wd_ref[0], preferred_element_type=jnp.float32)


def _moe(hidden_states, router_weight, gate_up_proj, down_proj, interpret=False):
    nf = FF // FB
    out, logits = pl.pallas_call(
        _moe_kernel,
        grid=(E, nf),
        in_specs=[
            pl.BlockSpec((T, H), lambda e, f: (0, 0)),
            pl.BlockSpec((E, H), lambda e, f: (0, 0)),
            pl.BlockSpec((1, H, FB), lambda e, f: (e, 0, f)),
            pl.BlockSpec((1, H, FB), lambda e, f: (e, 0, f + FF // FB)),
            pl.BlockSpec((1, FB, H), lambda e, f: (e, f, 0)),
        ],
        out_specs=[
            pl.BlockSpec((T, H), lambda e, f: (0, 0)),
            pl.BlockSpec((T, E), lambda e, f: (0, 0)),
        ],
        out_shape=[
            jax.ShapeDtypeStruct((T, H), jnp.float32),
            jax.ShapeDtypeStruct((T, E), jnp.float32),
        ],
        scratch_shapes=[pltpu.VMEM((T, E), jnp.float32)],
        interpret=interpret,
    )(hidden_states, router_weight, gate_up_proj, gate_up_proj, down_proj)
    return out, logits


def kernel(hidden_states, router_weight, gate_up_proj, down_proj):
    return _moe(hidden_states.reshape(-1, H), router_weight,
                gate_up_proj, down_proj)
